# Initial kernel scaffold; baseline (speedup 1.0000x reference)
import jax
import jax.numpy as jnp
from jax.experimental import pallas as pl


def kernel(x, edge_index, edge_feat, edge_list, W_np, b_np, W_en, b_en, b1, b2, W_p1, b_p1, W_p2, b_p2):
    raise NotImplementedError("write your pallas kernel here")


# speedup vs baseline: 3.3754x; 3.3754x over previous
import functools

import jax
import jax.numpy as jnp
from jax import lax
from jax.experimental import pallas as pl
from jax.experimental.pallas import tpu as pltpu
from jax.experimental.pallas import tpu_sc as plsc

N_NODES = 10000
N_EDGES = 160000
D_FEAT = 128
D_EDGE = 16
H = 16
N_PRED = 100000

_NC = 2
_NS = 16
_NW = _NC * _NS

_SC_PARAMS = pltpu.CompilerParams(use_tc_tiling_on_sc=False)


def _sc_gather(table, idx):
    B = idx.shape[0]
    b_per_w = B // _NW
    mesh = plsc.VectorSubcoreMesh(core_axis_name="c", subcore_axis_name="s")

    @functools.partial(
        pl.kernel, mesh=mesh, compiler_params=_SC_PARAMS,
        out_type=jax.ShapeDtypeStruct((B, H), jnp.float32),
        scratch_types=[
            pltpu.VMEM((b_per_w,), jnp.int32),
            pltpu.VMEM((b_per_w, H), jnp.float32),
            pltpu.SemaphoreType.DMA,
        ],
    )
    def k(table_hbm, idx_hbm, out_hbm, idx_v, rows_v, sem):
        wid = lax.axis_index("s") * _NC + lax.axis_index("c")
        base = wid * b_per_w
        pltpu.sync_copy(idx_hbm.at[pl.ds(base, b_per_w)], idx_v)
        pltpu.async_copy(table_hbm.at[idx_v], rows_v, sem).wait()
        pltpu.sync_copy(rows_v, out_hbm.at[pl.ds(base, b_per_w)])

    return k(table, idx)


def _sc_scatter_add(rows, idx, zeros):
    E = rows.shape[0]
    b_per_w = E // _NW
    npt = N_NODES // _NS
    mesh = plsc.VectorSubcoreMesh(core_axis_name="c", subcore_axis_name="s")

    @functools.partial(
        pl.kernel, mesh=mesh, compiler_params=_SC_PARAMS,
        out_type=jax.ShapeDtypeStruct((_NC * N_NODES, H), jnp.float32),
        scratch_types=[
            pltpu.VMEM((b_per_w,), jnp.int32),
            pltpu.VMEM((b_per_w, H), jnp.float32),
            pltpu.VMEM_SHARED((N_NODES, H), jnp.float32),
        ],
    )
    def k(rows_hbm, idx_hbm, zeros_hbm, out_hbm, idx_v, rows_v, acc):
        cid = lax.axis_index("c")
        sid = lax.axis_index("s")
        wid = sid * _NC + cid
        base = wid * b_per_w
        pltpu.sync_copy(idx_hbm.at[pl.ds(base, b_per_w)], idx_v)
        pltpu.sync_copy(rows_hbm.at[pl.ds(base, b_per_w)], rows_v)
        pltpu.sync_copy(zeros_hbm.at[pl.ds(sid * npt, npt)],
                        acc.at[pl.ds(sid * npt, npt)])
        plsc.subcore_barrier()
        pltpu.sync_copy(rows_v, acc.at[idx_v], add=True)
        plsc.subcore_barrier()
        pltpu.sync_copy(acc.at[pl.ds(sid * npt, npt)],
                        out_hbm.at[pl.ds(cid * N_NODES + sid * npt, npt)])

    return k(rows, idx, zeros)


def _tc_node_proj(x, W, b):
    def body(x_ref, w_ref, b_ref, o_ref):
        o_ref[...] = jnp.dot(x_ref[...], w_ref[...],
                             preferred_element_type=jnp.float32) + b_ref[...]
    return pl.pallas_call(
        body, out_shape=jax.ShapeDtypeStruct((N_NODES, H), jnp.float32),
    )(x, W, b)


def _tc_messages(ef, g, W_en, b_en, P, Q):
    E = ef.shape[0]
    Eb = 4000
    grid = E // Eb

    def body(ef_ref, g_ref, w_ref, b_ref, p_ref, q_ref, o_ref):
        we = jnp.maximum(
            jnp.dot(ef_ref[...], w_ref[...],
                    preferred_element_type=jnp.float32) + b_ref[...], 0.0)
        ge = jnp.dot(g_ref[...], p_ref[...], preferred_element_type=jnp.float32)
        o_ref[...] = jnp.dot(ge * we, q_ref[...],
                             preferred_element_type=jnp.float32)

    return pl.pallas_call(
        body, grid=(grid,),
        in_specs=[
            pl.BlockSpec((Eb, D_EDGE), lambda i: (i, 0)),
            pl.BlockSpec((Eb, H), lambda i: (i, 0)),
            pl.BlockSpec((D_EDGE, H * H), lambda i: (0, 0)),
            pl.BlockSpec((1, H * H), lambda i: (0, 0)),
            pl.BlockSpec((H, H * H), lambda i: (0, 0)),
            pl.BlockSpec((H * H, H), lambda i: (0, 0)),
        ],
        out_specs=pl.BlockSpec((Eb, H), lambda i: (i, 0)),
        out_shape=jax.ShapeDtypeStruct((E, H), jnp.float32),
    )(ef, g, W_en, b_en, P, Q)


def _tc_combine(p0, p1, c0, c1, b):
    def body(p0r, p1r, c0r, c1r, br, o_ref):
        s = p0r[...] + p1r[...]
        cnt = jnp.maximum(c0r[...] + c1r[...], 1.0)
        o_ref[...] = jnp.maximum(s / cnt + br[...], 0.0)
    return pl.pallas_call(
        body, out_shape=jax.ShapeDtypeStruct((N_NODES, H), jnp.float32),
    )(p0, p1, c0, c1, b)


def _tc_predict(sh, dh, Wa, Wb, bp1, Wp2, bp2):
    Eb = 4000
    grid = N_PRED // Eb

    def body(s_ref, d_ref, wa, wb, b1r, w2, b2r, o_ref):
        z = jnp.maximum(
            jnp.dot(s_ref[...], wa[...], preferred_element_type=jnp.float32)
            + jnp.dot(d_ref[...], wb[...], preferred_element_type=jnp.float32)
            + b1r[...], 0.0)
        o_ref[...] = jnp.dot(z, w2[...],
                             preferred_element_type=jnp.float32) + b2r[...]

    return pl.pallas_call(
        body, grid=(grid,),
        in_specs=[
            pl.BlockSpec((Eb, H), lambda i: (i, 0)),
            pl.BlockSpec((Eb, H), lambda i: (i, 0)),
            pl.BlockSpec((H, H), lambda i: (0, 0)),
            pl.BlockSpec((H, H), lambda i: (0, 0)),
            pl.BlockSpec((1, H), lambda i: (0, 0)),
            pl.BlockSpec((H, 1), lambda i: (0, 0)),
            pl.BlockSpec((1, 1), lambda i: (0, 0)),
        ],
        out_specs=pl.BlockSpec((Eb, 1), lambda i: (i, 0)),
        out_shape=jax.ShapeDtypeStruct((N_PRED, 1), jnp.float32),
    )(sh, dh, Wa, Wb, bp1, Wp2, bp2)


def kernel(x, edge_index, edge_feat, edge_list, W_np, b_np, W_en, b_en,
           b1, b2, W_p1, b_p1, W_p2, b_p2):
    src = edge_index[0]
    dst = edge_index[1]

    ii = jnp.arange(H * H)
    P = (jnp.arange(H)[:, None] == (ii[None, :] // H)).astype(jnp.float32)
    Q = ((ii[:, None] % H) == jnp.arange(H)[None, :]).astype(jnp.float32)
    zeros = jnp.zeros((N_NODES, H), jnp.float32)
    ones = jnp.ones((N_EDGES, H), jnp.float32)
    b_en2 = b_en.reshape(1, H * H)

    h0 = _tc_node_proj(x, W_np, b_np.reshape(1, H))
    cntp = _sc_scatter_add(ones, dst, zeros)

    g1 = _sc_gather(h0, src)
    m1 = _tc_messages(edge_feat, g1, W_en, b_en2, P, Q)
    s1 = _sc_scatter_add(m1, dst, zeros)
    h1 = _tc_combine(s1[:N_NODES], s1[N_NODES:], cntp[:N_NODES],
                     cntp[N_NODES:], b1.reshape(1, H))

    g2 = _sc_gather(h1, src)
    m2 = _tc_messages(edge_feat, g2, W_en, b_en2, P, Q)
    s2 = _sc_scatter_add(m2, dst, zeros)
    h2 = _tc_combine(s2[:N_NODES], s2[N_NODES:], cntp[:N_NODES],
                     cntp[N_NODES:], b2.reshape(1, H))

    pad = jnp.zeros((192,), jnp.int32)
    idx_pred = jnp.concatenate([edge_list[:, 0], edge_list[:, 1], pad])
    gp = _sc_gather(h2, idx_pred)
    logits = _tc_predict(gp[:N_PRED], gp[N_PRED:2 * N_PRED],
                         W_p1[:H], W_p1[H:], b_p1.reshape(1, H),
                         W_p2, b_p2.reshape(1, 1))
    return logits


# speedup vs baseline: 4.6474x; 1.3768x over previous
import functools

import jax
import jax.numpy as jnp
from jax import lax
from jax.experimental import pallas as pl
from jax.experimental.pallas import tpu as pltpu
from jax.experimental.pallas import tpu_sc as plsc

N_NODES = 10000
N_EDGES = 160000
D_FEAT = 128
D_EDGE = 16
H = 16
N_PRED = 100000
B_PRED = 204800

_NC = 2
_NS = 16
_NW = _NC * _NS

_SC_PARAMS = pltpu.CompilerParams(use_tc_tiling_on_sc=False)


def _sc_gather(table, idx):
    B = idx.shape[0]
    b_per_w = B // _NW
    mesh = plsc.VectorSubcoreMesh(core_axis_name="c", subcore_axis_name="s")

    @functools.partial(
        pl.kernel, mesh=mesh, compiler_params=_SC_PARAMS,
        out_type=jax.ShapeDtypeStruct((B, H), jnp.float32),
        scratch_types=[
            pltpu.VMEM((b_per_w,), jnp.int32),
            pltpu.VMEM((b_per_w, H), jnp.float32),
            pltpu.SemaphoreType.DMA,
        ],
    )
    def k(table_hbm, idx_hbm, out_hbm, idx_v, rows_v, sem):
        wid = lax.axis_index("s") * _NC + lax.axis_index("c")
        base = wid * b_per_w
        pltpu.sync_copy(idx_hbm.at[pl.ds(base, b_per_w)], idx_v)
        pltpu.async_copy(table_hbm.at[idx_v], rows_v, sem).wait()
        pltpu.sync_copy(rows_v, out_hbm.at[pl.ds(base, b_per_w)])

    return k(table, idx)


def _sc_scatter_add(rows, idx, zeros):
    E = rows.shape[0]
    b_per_w = E // _NW
    npt = N_NODES // _NS
    mesh = plsc.VectorSubcoreMesh(core_axis_name="c", subcore_axis_name="s")

    @functools.partial(
        pl.kernel, mesh=mesh, compiler_params=_SC_PARAMS,
        out_type=jax.ShapeDtypeStruct((_NC * N_NODES, H), jnp.float32),
        scratch_types=[
            pltpu.VMEM((b_per_w,), jnp.int32),
            pltpu.VMEM((b_per_w, H), jnp.float32),
            pltpu.VMEM_SHARED((N_NODES, H), jnp.float32),
        ],
    )
    def k(rows_hbm, idx_hbm, zeros_hbm, out_hbm, idx_v, rows_v, acc):
        cid = lax.axis_index("c")
        sid = lax.axis_index("s")
        wid = sid * _NC + cid
        base = wid * b_per_w
        pltpu.sync_copy(idx_hbm.at[pl.ds(base, b_per_w)], idx_v)
        pltpu.sync_copy(rows_hbm.at[pl.ds(base, b_per_w)], rows_v)
        pltpu.sync_copy(zeros_hbm.at[pl.ds(sid * npt, npt)],
                        acc.at[pl.ds(sid * npt, npt)])
        plsc.subcore_barrier()
        pltpu.sync_copy(rows_v, acc.at[idx_v], add=True)
        plsc.subcore_barrier()
        pltpu.sync_copy(acc.at[pl.ds(sid * npt, npt)],
                        out_hbm.at[pl.ds(cid * N_NODES + sid * npt, npt)])

    return k(rows, idx, zeros)


def _tc_node_proj(x, W, b):
    def body(x_ref, w_ref, b_ref, o_ref):
        o_ref[...] = jnp.dot(x_ref[...], w_ref[...],
                             preferred_element_type=jnp.float32) + b_ref[...]
    return pl.pallas_call(
        body, out_shape=jax.ShapeDtypeStruct((N_NODES, H), jnp.float32),
    )(x, W, b)


def _tc_messages(efp, gp, W8, b8, P8, Q8):
    Ep = efp.shape[0]
    Eb = 400
    grid = Ep // Eb

    def body(ef_ref, g_ref, w_ref, b_ref, p_ref, q_ref, o_ref):
        we = jnp.maximum(
            jnp.dot(ef_ref[...], w_ref[...],
                    preferred_element_type=jnp.float32) + b_ref[...], 0.0)
        ge = jnp.dot(g_ref[...], p_ref[...], preferred_element_type=jnp.float32)
        o_ref[...] = jnp.dot(ge * we, q_ref[...],
                             preferred_element_type=jnp.float32)

    return pl.pallas_call(
        body, grid=(grid,),
        in_specs=[
            pl.BlockSpec((Eb, 128), lambda i: (i, 0)),
            pl.BlockSpec((Eb, 128), lambda i: (i, 0)),
            pl.BlockSpec((128, 8 * H * H), lambda i: (0, 0)),
            pl.BlockSpec((1, 8 * H * H), lambda i: (0, 0)),
            pl.BlockSpec((128, 8 * H * H), lambda i: (0, 0)),
            pl.BlockSpec((8 * H * H, 128), lambda i: (0, 0)),
        ],
        out_specs=pl.BlockSpec((Eb, 128), lambda i: (i, 0)),
        out_shape=jax.ShapeDtypeStruct((Ep, 128), jnp.float32),
    )(efp, gp, W8, b8, P8, Q8)


def _tc_combine(p0, p1, c0, c1, b):
    def body(p0r, p1r, c0r, c1r, br, o_ref):
        s = p0r[...] + p1r[...]
        cnt = jnp.maximum(c0r[...] + c1r[...], 1.0)
        o_ref[...] = jnp.maximum(s / cnt + br[...], 0.0)
    return pl.pallas_call(
        body, out_shape=jax.ShapeDtypeStruct((N_NODES // 8, 128), jnp.float32),
    )(p0, p1, c0, c1, b)


def _tc_predict(shp, dhp, Wa8, Wb8, b18, W28, b28):
    Rp = shp.shape[0]
    Eb = 800
    grid = Rp // Eb

    def body(s_ref, d_ref, wa, wb, b1r, w2, b2r, o_ref):
        z = jnp.maximum(
            jnp.dot(s_ref[...], wa[...], preferred_element_type=jnp.float32)
            + jnp.dot(d_ref[...], wb[...], preferred_element_type=jnp.float32)
            + b1r[...], 0.0)
        o_ref[...] = jnp.dot(z, w2[...],
                             preferred_element_type=jnp.float32) + b2r[...]

    return pl.pallas_call(
        body, grid=(grid,),
        in_specs=[
            pl.BlockSpec((Eb, 128), lambda i: (i, 0)),
            pl.BlockSpec((Eb, 128), lambda i: (i, 0)),
            pl.BlockSpec((128, 128), lambda i: (0, 0)),
            pl.BlockSpec((128, 128), lambda i: (0, 0)),
            pl.BlockSpec((1, 128), lambda i: (0, 0)),
            pl.BlockSpec((128, 8), lambda i: (0, 0)),
            pl.BlockSpec((1, 8), lambda i: (0, 0)),
        ],
        out_specs=pl.BlockSpec((Eb, 8), lambda i: (i, 0)),
        out_shape=jax.ShapeDtypeStruct((Rp, 8), jnp.float32),
    )(shp, dhp, Wa8, Wb8, b18, W28, b28)


def _blockdiag8(W):
    r, c = W.shape
    big = jnp.tile(W, (8, 8))
    mask = jnp.kron(jnp.eye(8, dtype=W.dtype), jnp.ones((r, c), W.dtype))
    return big * mask


def kernel(x, edge_index, edge_feat, edge_list, W_np, b_np, W_en, b_en,
           b1, b2, W_p1, b_p1, W_p2, b_p2):
    f32 = jnp.float32
    src = edge_index[0]
    dst = edge_index[1]

    ii = jnp.arange(H * H)
    P = (jnp.arange(H)[:, None] == (ii[None, :] // H)).astype(f32)
    Q = ((ii[:, None] % H) == jnp.arange(H)[None, :]).astype(f32)
    P8 = _blockdiag8(P)
    Q8 = _blockdiag8(Q)
    W8 = _blockdiag8(W_en)
    b8 = jnp.tile(b_en, 8).reshape(1, 8 * H * H)
    zeros = jnp.zeros((N_NODES, H), f32)
    ones = jnp.ones((N_EDGES // 8, 128), f32).reshape(N_EDGES, H)
    efp = edge_feat.reshape(N_EDGES // 8, 128)

    h0 = _tc_node_proj(x, W_np, b_np.reshape(1, H))
    cntp = _sc_scatter_add(ones, dst, zeros)
    cnt0 = cntp[:N_NODES].reshape(N_NODES // 8, 128)
    cnt1 = cntp[N_NODES:].reshape(N_NODES // 8, 128)

    def conv(h_table, bias):
        g = _sc_gather(h_table, src)
        m = _tc_messages(efp, g.reshape(N_EDGES // 8, 128), W8, b8, P8, Q8)
        s = _sc_scatter_add(m.reshape(N_EDGES, H), dst, zeros)
        hp = _tc_combine(s[:N_NODES].reshape(N_NODES // 8, 128),
                         s[N_NODES:].reshape(N_NODES // 8, 128),
                         cnt0, cnt1, jnp.tile(bias, 8).reshape(1, 128))
        return hp.reshape(N_NODES, H)

    h1 = conv(h0, b1)
    h2 = conv(h1, b2)

    npad = B_PRED // 2 - N_PRED
    pad = jnp.zeros((npad,), jnp.int32)
    idx_pred = jnp.concatenate([edge_list[:, 0], pad, edge_list[:, 1], pad])
    gpk = _sc_gather(h2, idx_pred).reshape(B_PRED // 8, H * 8)
    half = B_PRED // 16
    logits8 = _tc_predict(gpk[:half], gpk[half:],
                          _blockdiag8(W_p1[:H]), _blockdiag8(W_p1[H:]),
                          jnp.tile(b_p1, 8).reshape(1, 128),
                          _blockdiag8(W_p2), jnp.tile(b_p2, 8).reshape(1, 8))
    return logits8.reshape(B_PRED // 2, 1)[:N_PRED]


# speedup vs baseline: 5.9102x; 1.2717x over previous
import functools

import jax
import jax.numpy as jnp
from jax import lax
from jax.experimental import pallas as pl
from jax.experimental.pallas import tpu as pltpu
from jax.experimental.pallas import tpu_sc as plsc

N_NODES = 10000
N_PAD = 10240
N_EDGES = 160000
D_FEAT = 128
D_EDGE = 16
H = 16
N_PRED = 100000
B_PRED = 204800

_NC = 2
_NS = 16
_NW = _NC * _NS

_SC_PARAMS = pltpu.CompilerParams(use_tc_tiling_on_sc=False)
_MESH = dict(core_axis_name="c", subcore_axis_name="s")


def _sc_gather(table, idx):
    B = idx.shape[0]
    b_per_w = B // _NW

    @functools.partial(
        pl.kernel, mesh=plsc.VectorSubcoreMesh(**_MESH),
        compiler_params=_SC_PARAMS,
        out_type=jax.ShapeDtypeStruct((B, H), jnp.float32),
        scratch_types=[
            pltpu.VMEM((b_per_w,), jnp.int32),
            pltpu.VMEM((b_per_w, H), jnp.float32),
            pltpu.SemaphoreType.DMA,
        ],
    )
    def k(table_hbm, idx_hbm, out_hbm, idx_v, rows_v, sem):
        wid = lax.axis_index("s") * _NC + lax.axis_index("c")
        base = wid * b_per_w
        pltpu.sync_copy(idx_hbm.at[pl.ds(base, b_per_w)], idx_v)
        pltpu.async_copy(table_hbm.at[idx_v], rows_v, sem).wait()
        pltpu.sync_copy(rows_v, out_hbm.at[pl.ds(base, b_per_w)])

    return k(table, idx)


def _sc_scatter_add(rows, idx, zeros, shared_rows=False):
    E = N_EDGES
    b_per_w = E // _NW
    npt = N_NODES // _NS

    @functools.partial(
        pl.kernel, mesh=plsc.VectorSubcoreMesh(**_MESH),
        compiler_params=_SC_PARAMS,
        out_type=jax.ShapeDtypeStruct((2 * N_NODES, H), jnp.float32),
        scratch_types=[
            pltpu.VMEM((b_per_w,), jnp.int32),
            pltpu.VMEM((b_per_w, H), jnp.float32),
            pltpu.VMEM_SHARED((N_NODES, H), jnp.float32),
        ],
    )
    def k(rows_hbm, idx_hbm, zeros_hbm, out_hbm, idx_v, rows_v, acc):
        cid = lax.axis_index("c")
        sid = lax.axis_index("s")
        wid = sid * _NC + cid
        base = wid * b_per_w
        pltpu.sync_copy(idx_hbm.at[pl.ds(base, b_per_w)], idx_v)
        if shared_rows:
            pltpu.sync_copy(rows_hbm, rows_v)
        else:
            pltpu.sync_copy(rows_hbm.at[pl.ds(base, b_per_w)], rows_v)
        pltpu.sync_copy(zeros_hbm.at[pl.ds(sid * npt, npt)],
                        acc.at[pl.ds(sid * npt, npt)])
        plsc.subcore_barrier()
        pltpu.sync_copy(rows_v, acc.at[idx_v], add=True)
        plsc.subcore_barrier()
        pltpu.sync_copy(acc.at[pl.ds(sid * npt, npt)],
                        out_hbm.at[pl.ds(cid * N_NODES + sid * npt, npt)])

    return k(rows, idx, zeros)


def _tc_node_proj(x, W, b):
    def body(x_ref, w_ref, b_ref, o_ref):
        o_ref[...] = jnp.dot(x_ref[...], w_ref[...],
                             preferred_element_type=jnp.float32) + b_ref[...]
    return pl.pallas_call(
        body, out_shape=jax.ShapeDtypeStruct((N_NODES, H), jnp.float32),
    )(x, W, b)


def _tc_messages(efp, gp, W8, b8, P8, Q8):
    Ep = efp.shape[0]
    Eb = 400
    grid = Ep // Eb
    bf16 = jnp.bfloat16

    def body(ef_ref, g_ref, w_ref, b_ref, p_ref, q_ref, o_ref):
        we = jnp.maximum(
            jnp.dot(ef_ref[...].astype(bf16), w_ref[...],
                    preferred_element_type=jnp.float32) + b_ref[...], 0.0)
        ge = jnp.dot(g_ref[...].astype(bf16), p_ref[...],
                     preferred_element_type=jnp.float32)
        t = (ge * we).astype(bf16)
        o_ref[...] = jnp.dot(t, q_ref[...], preferred_element_type=jnp.float32)

    return pl.pallas_call(
        body, grid=(grid,),
        in_specs=[
            pl.BlockSpec((Eb, 128), lambda i: (i, 0)),
            pl.BlockSpec((Eb, 128), lambda i: (i, 0)),
            pl.BlockSpec((128, 8 * H * H), lambda i: (0, 0)),
            pl.BlockSpec((1, 8 * H * H), lambda i: (0, 0)),
            pl.BlockSpec((128, 8 * H * H), lambda i: (0, 0)),
            pl.BlockSpec((8 * H * H, 128), lambda i: (0, 0)),
        ],
        out_specs=pl.BlockSpec((Eb, 128), lambda i: (i, 0)),
        out_shape=jax.ShapeDtypeStruct((Ep, 128), jnp.float32),
    )(efp, gp, W8, b8, P8, Q8)


def _tc_combine(p0, p1, c0, c1, b):
    def body(p0r, p1r, c0r, c1r, br, o_ref):
        s = p0r[...] + p1r[...]
        cnt = jnp.maximum(c0r[...] + c1r[...], 1.0)
        o_ref[...] = jnp.maximum(s / cnt + br[...], 0.0)
    return pl.pallas_call(
        body, out_shape=jax.ShapeDtypeStruct((N_NODES // 8, 128), jnp.float32),
    )(p0, p1, c0, c1, b)


def _tc_predict(gpk, Wa8, Wb8, b18, W28, b28):
    Rp = gpk.shape[0] // 2
    Eb = 800
    grid = Rp // Eb
    off = Rp // Eb

    def body(s_ref, d_ref, wa, wb, b1r, w2, b2r, o_ref):
        z = jnp.maximum(
            jnp.dot(s_ref[...], wa[...], preferred_element_type=jnp.float32)
            + jnp.dot(d_ref[...], wb[...], preferred_element_type=jnp.float32)
            + b1r[...], 0.0)
        o_ref[...] = jnp.dot(z, w2[...],
                             preferred_element_type=jnp.float32) + b2r[...]

    return pl.pallas_call(
        body, grid=(grid,),
        in_specs=[
            pl.BlockSpec((Eb, 128), lambda i: (i, 0)),
            pl.BlockSpec((Eb, 128), lambda i, off=off: (i + off, 0)),
            pl.BlockSpec((128, 128), lambda i: (0, 0)),
            pl.BlockSpec((128, 128), lambda i: (0, 0)),
            pl.BlockSpec((1, 128), lambda i: (0, 0)),
            pl.BlockSpec((128, 8), lambda i: (0, 0)),
            pl.BlockSpec((1, 8), lambda i: (0, 0)),
        ],
        out_specs=pl.BlockSpec((Eb, 8), lambda i: (i, 0)),
        out_shape=jax.ShapeDtypeStruct((Rp, 8), jnp.float32),
    )(gpk, gpk, Wa8, Wb8, b18, W28, b28)


def _blockdiag8(W):
    r, c = W.shape
    big = jnp.tile(W, (8, 8))
    mask = jnp.kron(jnp.eye(8, dtype=W.dtype), jnp.ones((r, c), W.dtype))
    return big * mask


def kernel(x, edge_index, edge_feat, edge_list, W_np, b_np, W_en, b_en,
           b1, b2, W_p1, b_p1, W_p2, b_p2):
    f32 = jnp.float32
    bf16 = jnp.bfloat16
    src = edge_index[0]
    dst = edge_index[1]

    ii = jnp.arange(H * H)
    P = (jnp.arange(H)[:, None] == (ii[None, :] // H)).astype(bf16)
    Q = ((ii[:, None] % H) == jnp.arange(H)[None, :]).astype(bf16)
    P8 = _blockdiag8(P)
    Q8 = _blockdiag8(Q)
    W8 = _blockdiag8(W_en.astype(bf16))
    b8 = jnp.tile(b_en, 8).reshape(1, 8 * H * H)
    zeros = jnp.zeros((N_NODES, H), f32)
    ones = jnp.ones((N_EDGES // _NW, H), f32)
    efp = edge_feat.reshape(N_EDGES // 8, 128)
    pk = (N_NODES // 8, 128)

    h0 = _tc_node_proj(x, W_np, b_np.reshape(1, H))
    cntp = _sc_scatter_add(ones, dst, zeros, shared_rows=True)
    cnt0 = cntp[:N_NODES].reshape(pk)
    cnt1 = cntp[N_NODES:].reshape(pk)

    def conv(h_table, bias):
        g = _sc_gather(h_table, src)
        m = _tc_messages(efp, g.reshape(N_EDGES // 8, 128), W8, b8, P8, Q8)
        s = _sc_scatter_add(m.reshape(N_EDGES, H), dst, zeros)
        hp = _tc_combine(s[:N_NODES].reshape(pk), s[N_NODES:].reshape(pk),
                         cnt0, cnt1, jnp.tile(bias, 8).reshape(1, 128))
        return hp.reshape(N_NODES, H)

    h1 = conv(h0, b1)
    h2 = conv(h1, b2)

    npad = B_PRED // 2 - N_PRED
    pad = jnp.zeros((npad,), jnp.int32)
    idx_pred = jnp.concatenate([edge_list[:, 0], pad, edge_list[:, 1], pad])
    gpk = _sc_gather(h2, idx_pred).reshape(B_PRED // 8, 128)
    logits8 = _tc_predict(gpk,
                          _blockdiag8(W_p1[:H]), _blockdiag8(W_p1[H:]),
                          jnp.tile(b_p1, 8).reshape(1, 128),
                          _blockdiag8(W_p2), jnp.tile(b_p2, 8).reshape(1, 8))
    return logits8.reshape(B_PRED // 2, 1)[:N_PRED]
